# Initial kernel scaffold; baseline (speedup 1.0000x reference)
import jax
import jax.numpy as jnp
from jax.experimental import pallas as pl


def kernel(adj1, adj2, W0, b0, Ws, bs, Wl, bl, Wm1, bm1, Wm2, bm2):
    raise NotImplementedError("write your pallas kernel here")


# speedup vs baseline: 1.5439x; 1.5439x over previous
import jax
import jax.numpy as jnp
from jax.experimental import pallas as pl
from jax.experimental.pallas import tpu as pltpu

N = 10000
H = 128
BM = 400

_CP = pltpu.CompilerParams(vmem_limit_bytes=100 * 1024 * 1024)


def _prop_body(adj_ref, y_ref, b_ref, wn_ref, out_ref):
    z = jnp.dot(adj_ref[...], y_ref[...], preferred_element_type=jnp.float32)
    z = jnp.maximum(z + b_ref[...], 0.0)
    n = jnp.sqrt(jnp.sum(z * z, axis=1, keepdims=True))
    x = z / jnp.maximum(n, 1e-12)
    out_ref[...] = jnp.dot(x, wn_ref[...], preferred_element_type=jnp.float32)


def _prop_keep_body(adj_ref, y_ref, b_ref, wn_ref, x_ref, out_ref):
    z = jnp.dot(adj_ref[...], y_ref[...], preferred_element_type=jnp.float32)
    z = jnp.maximum(z + b_ref[...], 0.0)
    n = jnp.sqrt(jnp.sum(z * z, axis=1, keepdims=True))
    x = z / jnp.maximum(n, 1e-12)
    x_ref[...] = x
    out_ref[...] = jnp.dot(x, wn_ref[...], preferred_element_type=jnp.float32)


def _score_body(adj_ref, yc_ref, x_ref, bc_ref, wm1_ref, bm1_ref, wm2_ref,
                bm2_ref, out_ref):
    z = jnp.dot(adj_ref[...], yc_ref[...], preferred_element_type=jnp.float32)
    z = z + bc_ref[...]
    wm1 = wm1_ref[...]
    bm1 = bm1_ref[...]
    acc = jnp.maximum(
        jnp.dot(x_ref[...], wm1, preferred_element_type=jnp.float32) + bm1, 0.0)
    for k in range(4):
        h = jnp.maximum(z[:, k * H:(k + 1) * H], 0.0)
        nn = jnp.sqrt(jnp.sum(h * h, axis=1, keepdims=True))
        h = h / jnp.maximum(nn, 1e-12)
        acc = acc + jnp.maximum(
            jnp.dot(h, wm1, preferred_element_type=jnp.float32) + bm1, 0.0)
    hl = jnp.maximum(z[:, 4 * H:], 0.0)
    acc = acc + jnp.maximum(
        jnp.dot(hl, wm1, preferred_element_type=jnp.float32) + bm1, 0.0)
    out_ref[...] = (jnp.dot(acc, wm2_ref[...], preferred_element_type=jnp.float32)
                    + 6.0 * bm2_ref[...])


def _prop(adj, y, b, wn, keep_x=False):
    wout = wn.shape[1]
    grid = (N // BM,)
    in_specs = [
        pl.BlockSpec((BM, N), lambda i: (i, 0)),
        pl.BlockSpec((N, y.shape[1]), lambda i: (0, 0)),
        pl.BlockSpec((1, H), lambda i: (0, 0)),
        pl.BlockSpec((H, wout), lambda i: (0, 0)),
    ]
    if keep_x:
        return pl.pallas_call(
            _prop_keep_body,
            grid=grid,
            in_specs=in_specs,
            out_specs=[pl.BlockSpec((BM, H), lambda i: (i, 0)),
                       pl.BlockSpec((BM, wout), lambda i: (i, 0))],
            out_shape=[jax.ShapeDtypeStruct((N, H), jnp.float32),
                       jax.ShapeDtypeStruct((N, wout), jnp.float32)],
            compiler_params=_CP,
        )(adj, y, b.reshape(1, H), wn)
    return pl.pallas_call(
        _prop_body,
        grid=grid,
        in_specs=in_specs,
        out_specs=pl.BlockSpec((BM, wout), lambda i: (i, 0)),
        out_shape=jax.ShapeDtypeStruct((N, wout), jnp.float32),
        compiler_params=_CP,
    )(adj, y, b.reshape(1, H), wn)


def kernel(adj1, adj2, W0, b0, Ws, bs, Wl, bl, Wm1, bm1, Wm2, bm2):
    y = _prop(adj1, W0, b0, Ws[0])
    y = _prop(adj2, y, bs[0], Ws[1])
    y = _prop(adj2, y, bs[1], Ws[2])
    y = _prop(adj2, y, bs[2], Ws[3])
    Wcat = jnp.concatenate([Ws[0], Ws[1], Ws[2], Ws[3], Wl], axis=1)
    x4, yc = _prop(adj2, y, bs[3], Wcat, keep_x=True)
    bcat = jnp.concatenate([bs[0], bs[1], bs[2], bs[3], bl]).reshape(1, 5 * H)

    grid = (N // BM,)
    score = pl.pallas_call(
        _score_body,
        grid=grid,
        in_specs=[
            pl.BlockSpec((BM, N), lambda i: (i, 0)),
            pl.BlockSpec((N, 5 * H), lambda i: (0, 0)),
            pl.BlockSpec((BM, H), lambda i: (i, 0)),
            pl.BlockSpec((1, 5 * H), lambda i: (0, 0)),
            pl.BlockSpec((H, H), lambda i: (0, 0)),
            pl.BlockSpec((1, H), lambda i: (0, 0)),
            pl.BlockSpec((H, 1), lambda i: (0, 0)),
            pl.BlockSpec((1, 1), lambda i: (0, 0)),
        ],
        out_specs=pl.BlockSpec((BM, 1), lambda i: (i, 0)),
        out_shape=jax.ShapeDtypeStruct((N, 1), jnp.float32),
        compiler_params=_CP,
    )(adj2, yc, x4, bcat, Wm1, bm1.reshape(1, H), Wm2, bm2.reshape(1, 1))
    return score


# speedup vs baseline: 1.7116x; 1.1086x over previous
import jax
import jax.numpy as jnp
from jax.experimental import pallas as pl
from jax.experimental.pallas import tpu as pltpu

N = 10000
H = 128
BM = 400

_CP = pltpu.CompilerParams(vmem_limit_bytes=100 * 1024 * 1024)


def _relu_norm(z):
    z = jnp.maximum(z, 0.0)
    n = jnp.sqrt(jnp.sum(z * z, axis=1, keepdims=True))
    return z / jnp.maximum(n, 1e-12)


def _prop_body(adj_ref, y_ref, b_ref, wn_ref, out_ref):
    z = jnp.dot(adj_ref[...], y_ref[...], preferred_element_type=jnp.float32)
    x = _relu_norm(z + b_ref[...])
    out = jnp.dot(x, wn_ref[...], preferred_element_type=jnp.float32)
    out_ref[...] = out.astype(out_ref.dtype)


def _prop_cast_body(adj_ref, y_ref, b_ref, wn_ref, out_ref, adjb_ref):
    adj = adj_ref[...]
    adjb_ref[...] = adj.astype(jnp.bfloat16)
    z = jnp.dot(adj, y_ref[...], preferred_element_type=jnp.float32)
    x = _relu_norm(z + b_ref[...])
    out = jnp.dot(x, wn_ref[...], preferred_element_type=jnp.float32)
    out_ref[...] = out.astype(out_ref.dtype)


def _prop_keep_body(adj_ref, y_ref, b_ref, wn_ref, x_ref, out_ref):
    z = jnp.dot(adj_ref[...], y_ref[...], preferred_element_type=jnp.float32)
    x = _relu_norm(z + b_ref[...])
    x_ref[...] = x
    out = jnp.dot(x, wn_ref[...], preferred_element_type=jnp.float32)
    out_ref[...] = out.astype(out_ref.dtype)


def _score_body(adj_ref, yc_ref, x_ref, bc_ref, wm1_ref, bm1_ref, wm2_ref,
                bm2_ref, out_ref):
    z = jnp.dot(adj_ref[...], yc_ref[...], preferred_element_type=jnp.float32)
    z = z + bc_ref[...]
    wm1 = wm1_ref[...]
    bm1 = bm1_ref[...]
    acc = jnp.maximum(
        jnp.dot(x_ref[...], wm1, preferred_element_type=jnp.float32) + bm1, 0.0)
    for k in range(4):
        h = _relu_norm(z[:, k * H:(k + 1) * H])
        acc = acc + jnp.maximum(
            jnp.dot(h, wm1, preferred_element_type=jnp.float32) + bm1, 0.0)
    hl = jnp.maximum(z[:, 4 * H:], 0.0)
    acc = acc + jnp.maximum(
        jnp.dot(hl, wm1, preferred_element_type=jnp.float32) + bm1, 0.0)
    out_ref[...] = (jnp.dot(acc, wm2_ref[...], preferred_element_type=jnp.float32)
                    + 6.0 * bm2_ref[...])


def _in_specs(y_cols, wout):
    return [
        pl.BlockSpec((BM, N), lambda i: (i, 0)),
        pl.BlockSpec((N, y_cols), lambda i: (0, 0)),
        pl.BlockSpec((1, H), lambda i: (0, 0)),
        pl.BlockSpec((H, wout), lambda i: (0, 0)),
    ]


def _prop(adj, y, b, wn, out_dtype, body=_prop_body, extra_outs=()):
    wout = wn.shape[1]
    grid = (N // BM,)
    out_specs = [pl.BlockSpec((BM, wout), lambda i: (i, 0))]
    out_shape = [jax.ShapeDtypeStruct((N, wout), out_dtype)]
    for blk, shape, dt in extra_outs:
        out_specs.append(pl.BlockSpec(blk, lambda i: (i, 0)))
        out_shape.append(jax.ShapeDtypeStruct(shape, dt))
    res = pl.pallas_call(
        body,
        grid=grid,
        in_specs=_in_specs(y.shape[1], wout),
        out_specs=out_specs if len(out_specs) > 1 else out_specs[0],
        out_shape=out_shape if len(out_shape) > 1 else out_shape[0],
        compiler_params=_CP,
    )(adj, y, b.reshape(1, H), wn)
    return res


def kernel(adj1, adj2, W0, b0, Ws, bs, Wl, bl, Wm1, bm1, Wm2, bm2):
    bf16 = jnp.bfloat16
    y = _prop(adj1, W0, b0, Ws[0], jnp.float32)
    y, adj2b = _prop(adj2, y, bs[0], Ws[1].astype(bf16), bf16,
                     body=_prop_cast_body,
                     extra_outs=(((BM, N), (N, N), bf16),))
    y = _prop(adj2b, y, bs[1], Ws[2].astype(bf16), bf16)
    y = _prop(adj2b, y, bs[2], Ws[3].astype(bf16), bf16)
    Wcat = jnp.concatenate([Ws[0], Ws[1], Ws[2], Ws[3], Wl], axis=1)
    x4, yc = pl.pallas_call(
        _prop_keep_body,
        grid=(N // BM,),
        in_specs=_in_specs(H, 5 * H),
        out_specs=[pl.BlockSpec((BM, H), lambda i: (i, 0)),
                   pl.BlockSpec((BM, 5 * H), lambda i: (i, 0))],
        out_shape=[jax.ShapeDtypeStruct((N, H), jnp.float32),
                   jax.ShapeDtypeStruct((N, 5 * H), bf16)],
        compiler_params=_CP,
    )(adj2b, y, bs[3].reshape(1, H), Wcat.astype(bf16))
    bcat = jnp.concatenate([bs[0], bs[1], bs[2], bs[3], bl]).reshape(1, 5 * H)

    grid = (N // BM,)
    score = pl.pallas_call(
        _score_body,
        grid=grid,
        in_specs=[
            pl.BlockSpec((BM, N), lambda i: (i, 0)),
            pl.BlockSpec((N, 5 * H), lambda i: (0, 0)),
            pl.BlockSpec((BM, H), lambda i: (i, 0)),
            pl.BlockSpec((1, 5 * H), lambda i: (0, 0)),
            pl.BlockSpec((H, H), lambda i: (0, 0)),
            pl.BlockSpec((1, H), lambda i: (0, 0)),
            pl.BlockSpec((H, 1), lambda i: (0, 0)),
            pl.BlockSpec((1, 1), lambda i: (0, 0)),
        ],
        out_specs=pl.BlockSpec((BM, 1), lambda i: (i, 0)),
        out_shape=jax.ShapeDtypeStruct((N, 1), jnp.float32),
        compiler_params=_CP,
    )(adj2b, yc, x4, bcat, Wm1, bm1.reshape(1, H), Wm2, bm2.reshape(1, 1))
    return score


# speedup vs baseline: 1.7476x; 1.0210x over previous
import jax
import jax.numpy as jnp
from jax.experimental import pallas as pl
from jax.experimental.pallas import tpu as pltpu

N = 10000
H = 128
BM = 400
BL = 1000

_CP = pltpu.CompilerParams(vmem_limit_bytes=100 * 1024 * 1024)


def _relu_norm(z):
    z = jnp.maximum(z, 0.0)
    n = jnp.sqrt(jnp.sum(z * z, axis=1, keepdims=True))
    return z / jnp.maximum(n, 1e-12)


def _prop_body(adj_ref, y_ref, b_ref, wn_ref, out_ref):
    z = jnp.dot(adj_ref[...], y_ref[...], preferred_element_type=jnp.float32)
    x = _relu_norm(z + b_ref[...])
    out = jnp.dot(x, wn_ref[...], preferred_element_type=jnp.float32)
    out_ref[...] = out.astype(out_ref.dtype)


def _prop_cast_body(adj_ref, y_ref, b_ref, wn_ref, out_ref, adjb_ref):
    adj = adj_ref[...]
    adjb_ref[...] = adj.astype(jnp.bfloat16)
    z = jnp.dot(adj, y_ref[...], preferred_element_type=jnp.float32)
    x = _relu_norm(z + b_ref[...])
    out = jnp.dot(x, wn_ref[...], preferred_element_type=jnp.float32)
    out_ref[...] = out.astype(out_ref.dtype)


def _prop_keep_body(adj_ref, y_ref, b_ref, wn_ref, x_ref, out_ref):
    z = jnp.dot(adj_ref[...], y_ref[...], preferred_element_type=jnp.float32)
    x = _relu_norm(z + b_ref[...])
    x_ref[...] = x
    out = jnp.dot(x, wn_ref[...], preferred_element_type=jnp.float32)
    out_ref[...] = out.astype(out_ref.dtype)


def _score_body(adj_ref, yc_ref, x_ref, bc_ref, wm1_ref, bm1_ref, wm2_ref,
                bm2_ref, out_ref):
    z = jnp.dot(adj_ref[...], yc_ref[...], preferred_element_type=jnp.float32)
    z = z + bc_ref[...]
    wm1 = wm1_ref[...]
    bm1 = bm1_ref[...]
    acc = jnp.maximum(
        jnp.dot(x_ref[...], wm1, preferred_element_type=jnp.float32) + bm1, 0.0)
    for k in range(4):
        h = _relu_norm(z[:, k * H:(k + 1) * H])
        acc = acc + jnp.maximum(
            jnp.dot(h, wm1, preferred_element_type=jnp.float32) + bm1, 0.0)
    hl = jnp.maximum(z[:, 4 * H:], 0.0)
    acc = acc + jnp.maximum(
        jnp.dot(hl, wm1, preferred_element_type=jnp.float32) + bm1, 0.0)
    out_ref[...] = (jnp.dot(acc, wm2_ref[...], preferred_element_type=jnp.float32)
                    + 6.0 * bm2_ref[...])


def _in_specs(y_cols, wout, bm):
    return [
        pl.BlockSpec((bm, N), lambda i: (i, 0)),
        pl.BlockSpec((N, y_cols), lambda i: (0, 0)),
        pl.BlockSpec((1, H), lambda i: (0, 0)),
        pl.BlockSpec((H, wout), lambda i: (0, 0)),
    ]


def _prop(adj, y, b, wn, out_dtype, body=_prop_body, extra_outs=(), bm=None):
    if bm is None:
        bm = BM
    wout = wn.shape[1]
    grid = (N // bm,)
    out_specs = [pl.BlockSpec((bm, wout), lambda i: (i, 0))]
    out_shape = [jax.ShapeDtypeStruct((N, wout), out_dtype)]
    for blk, shape, dt in extra_outs:
        out_specs.append(pl.BlockSpec(blk, lambda i: (i, 0)))
        out_shape.append(jax.ShapeDtypeStruct(shape, dt))
    res = pl.pallas_call(
        body,
        grid=grid,
        in_specs=_in_specs(y.shape[1], wout, bm),
        out_specs=out_specs if len(out_specs) > 1 else out_specs[0],
        out_shape=out_shape if len(out_shape) > 1 else out_shape[0],
        compiler_params=_CP,
    )(adj, y, b.reshape(1, H), wn)
    return res


def kernel(adj1, adj2, W0, b0, Ws, bs, Wl, bl, Wm1, bm1, Wm2, bm2):
    bf16 = jnp.bfloat16
    y = _prop(adj1, W0, b0, Ws[0], jnp.float32)
    y, adj2b = _prop(adj2, y, bs[0], Ws[1].astype(bf16), bf16,
                     body=_prop_cast_body,
                     extra_outs=(((BM, N), (N, N), bf16),))
    y = _prop(adj2b, y, bs[1], Ws[2].astype(bf16), bf16, bm=BL)
    y = _prop(adj2b, y, bs[2], Ws[3].astype(bf16), bf16, bm=BL)
    Wcat = jnp.concatenate([Ws[0], Ws[1], Ws[2], Ws[3], Wl], axis=1)
    x4, yc = pl.pallas_call(
        _prop_keep_body,
        grid=(N // BL,),
        in_specs=_in_specs(H, 5 * H, BL),
        out_specs=[pl.BlockSpec((BL, H), lambda i: (i, 0)),
                   pl.BlockSpec((BL, 5 * H), lambda i: (i, 0))],
        out_shape=[jax.ShapeDtypeStruct((N, H), jnp.float32),
                   jax.ShapeDtypeStruct((N, 5 * H), bf16)],
        compiler_params=_CP,
    )(adj2b, y, bs[3].reshape(1, H), Wcat.astype(bf16))
    bcat = jnp.concatenate([bs[0], bs[1], bs[2], bs[3], bl]).reshape(1, 5 * H)

    score = pl.pallas_call(
        _score_body,
        grid=(N // BL,),
        in_specs=[
            pl.BlockSpec((BL, N), lambda i: (i, 0)),
            pl.BlockSpec((N, 5 * H), lambda i: (0, 0)),
            pl.BlockSpec((BL, H), lambda i: (i, 0)),
            pl.BlockSpec((1, 5 * H), lambda i: (0, 0)),
            pl.BlockSpec((H, H), lambda i: (0, 0)),
            pl.BlockSpec((1, H), lambda i: (0, 0)),
            pl.BlockSpec((H, 1), lambda i: (0, 0)),
            pl.BlockSpec((1, 1), lambda i: (0, 0)),
        ],
        out_specs=pl.BlockSpec((BL, 1), lambda i: (i, 0)),
        out_shape=jax.ShapeDtypeStruct((N, 1), jnp.float32),
        compiler_params=_CP,
    )(adj2b, yc, x4, bcat, Wm1, bm1.reshape(1, H), Wm2, bm2.reshape(1, 1))
    return score
